# Initial kernel scaffold; baseline (speedup 1.0000x reference)
import jax
import jax.numpy as jnp
from jax.experimental import pallas as pl


def kernel(x, coordinates):
    raise NotImplementedError("write your pallas kernel here")


# speedup vs baseline: 5.0011x; 5.0011x over previous
import functools

import jax
import jax.numpy as jnp
from jax import lax
from jax.experimental import pallas as pl
from jax.experimental.pallas import tpu as pltpu
from jax.experimental.pallas import tpu_sc as plsc

_F = 8192
_K = 8
_B = 1024
_CD = 64

_ROWS = 128


_DEPTH = 4
_LANES = 128
_BIGI = 2 ** 30


def _extract8(vals, idxs):
    cols = []
    m = None
    for _ in range(_K):
        m = jnp.min(vals, axis=1, keepdims=True)
        cand = jnp.where(vals == m, idxs, jnp.int32(_BIGI))
        sel = jnp.min(cand, axis=1, keepdims=True)
        cols.append(sel)
        vals = jnp.where(idxs == sel, jnp.float32(jnp.inf), vals)
    return jnp.concatenate(cols, axis=1), m


def _nbr_body(crd_ref, crdT_ref, idx_ref):
    a = crd_ref[...]
    bT = crdT_ref[...]
    na = jnp.sum(a * a, axis=1, keepdims=True)
    nb = jnp.sum(bT * bT, axis=0, keepdims=True)
    d = jnp.dot(a, bT, preferred_element_type=jnp.float32)
    d = -2.0 * d + nb
    d = d + na
    d = jnp.maximum(d, 0.0)
    r = d.shape[0]

    lane = lax.broadcasted_iota(jnp.int32, (r, _LANES), 1)
    ms = [jnp.full((r, _LANES), jnp.float32(jnp.inf))] * _DEPTH
    is_ = [jnp.full((r, _LANES), _BIGI, jnp.int32)] * _DEPTH
    for c in range(_F // _LANES):
        cv = lax.slice(d, (0, c * _LANES), (r, (c + 1) * _LANES))
        ci = lane + jnp.int32(c * _LANES)
        for lvl in range(_DEPTH):
            lt = cv < ms[lvl]
            nm = jnp.where(lt, cv, ms[lvl])
            ni = jnp.where(lt, ci, is_[lvl])
            if lvl + 1 < _DEPTH:
                cv = jnp.where(lt, ms[lvl], cv)
                ci = jnp.where(lt, is_[lvl], ci)
            ms[lvl] = nm
            is_[lvl] = ni

    pool = jnp.concatenate(ms, axis=1)
    pidx = jnp.concatenate(is_, axis=1)
    fast_idx, v8 = _extract8(pool, pidx)

    viol = jnp.any(ms[_DEPTH - 1] <= v8)

    @pl.when(jnp.logical_not(viol))
    def _():
        idx_ref[...] = fast_idx

    @pl.when(viol)
    def _():
        colid = lax.broadcasted_iota(jnp.int32, d.shape, 1)
        slow_idx, _unused = _extract8(d, colid)
        idx_ref[...] = slow_idx


def _neighbor_topk(crd, crdT, interpret=False):
    return pl.pallas_call(
        _nbr_body,
        grid=(_F // _ROWS,),
        in_specs=[
            pl.BlockSpec((_ROWS, _CD), lambda i: (i, 0)),
            pl.BlockSpec((_CD, _F), lambda i: (0, 0)),
        ],
        out_specs=pl.BlockSpec((_ROWS, _K), lambda i: (i, 0)),
        out_shape=jax.ShapeDtypeStruct((_F, _K), jnp.int32),
        interpret=interpret,
    )(crd, crdT)


def _tr_body(in_ref, out_ref):
    out_ref[...] = in_ref[...].T


def _transpose(mat, bi, bj, interpret=False):
    m, n = mat.shape
    return pl.pallas_call(
        _tr_body,
        grid=(m // bi, n // bj),
        in_specs=[pl.BlockSpec((bi, bj), lambda i, j: (i, j))],
        out_specs=pl.BlockSpec((bj, bi), lambda i, j: (j, i)),
        out_shape=jax.ShapeDtypeStruct((n, m), mat.dtype),
        interpret=interpret,
    )(mat)


_NC, _NS = 2, 16
_NW = _NC * _NS
_BPW = _F * _K // _NW
_CH = 64
_NCH = _BPW // _CH


def _gather_sc_body(table_hbm, idx_hbm, out_hbm, idx_v, rows_v, sem):
    wid = lax.axis_index("s") * _NC + lax.axis_index("c")
    base = pl.multiple_of(wid * _BPW, _BPW)
    pltpu.sync_copy(idx_hbm.at[pl.ds(base, _BPW)], idx_v)

    def chunk(i, carry):
        off = pl.multiple_of(i * _CH, _CH)
        cp = pltpu.async_copy(
            table_hbm.at[idx_v.at[pl.ds(off, _CH)]], rows_v, sem)
        cp.wait()
        pltpu.sync_copy(rows_v, out_hbm.at[pl.ds(base + off, _CH)])
        return carry

    lax.fori_loop(0, _NCH, chunk, 0)


@functools.cache
def _gather_sc():
    return pl.kernel(
        _gather_sc_body,
        out_type=jax.ShapeDtypeStruct((_F * _K, _B), jnp.float32),
        mesh=plsc.VectorSubcoreMesh(core_axis_name="c", subcore_axis_name="s"),
        scratch_types=[
            pltpu.VMEM((_BPW,), jnp.int32),
            pltpu.VMEM((_CH, _B), jnp.float32),
            pltpu.SemaphoreType.DMA,
        ],
    )


def kernel(x, coordinates):
    x2 = x.reshape(_B, _F)
    crdT = coordinates.reshape(_CD, _F)
    crd = crdT.T
    nbr = _neighbor_topk(crd, crdT)
    idx_flat = nbr.reshape(_F * _K)
    xT = _transpose(x2, 512, 512)
    outT = _gather_sc()(xT, idx_flat)
    out = _transpose(outT, 512, 1024)
    return out.reshape(_B, 1, _F * _K, 1)
